# baseline (speedup 1.0000x reference)
import functools

import jax
import jax.numpy as jnp
from jax import lax
from jax.experimental import pallas as pl
from jax.experimental.pallas import tpu as pltpu
from jax.experimental.pallas import tpu_sc as plsc

VOCAB = 1000000
D = 200
DP = 208
L = 200
B = 4096
NLANE = 16
NCHUNK = 13
TCH = 13
T_OFFS = tuple(list(range(0, 192, 16)) + [184])
NW = 32
BPW = B // NW
WSTRIDE = NCHUNK * NLANE


def _sc_logits_kernel(idx_hbm, table_hbm, w_hbm, c0_hbm, out_hbm,
                      idx_v, rows_v, w_v, c0_v, log_v, sem, gsem):
  wid = lax.axis_index("s") * 2 + lax.axis_index("c")
  base = wid * BPW
  pltpu.sync_copy(w_hbm, w_v)
  pltpu.sync_copy(c0_hbm, c0_v)

  def per_elem(b_local, carry):
    b_abs = base + b_local
    pltpu.sync_copy(idx_hbm.at[b_abs], idx_v)
    cp0 = pltpu.async_copy(table_hbm.at[idx_v.at[pl.ds(0, 104)]],
                           rows_v.at[pl.ds(0, 104)], sem)
    cp1 = pltpu.async_copy(table_hbm.at[idx_v.at[pl.ds(104, 96)]],
                           rows_v.at[pl.ds(104, 96)], gsem)
    cp0.wait()
    cp1.wait()

    def per_t(t, accs):
      a0, a1 = accs
      wbase = t * (2 * WSTRIDE)
      for k in range(NCHUNK):
        off = k * NLANE
        r = rows_v[t, pl.ds(off, NLANE)]
        a0 = a0 + r * w_v[pl.ds(wbase + off, NLANE)]
        a1 = a1 + r * w_v[pl.ds(wbase + WSTRIDE + off, NLANE)]
      return (a0, a1)

    z = jnp.zeros((NLANE,), jnp.float32)
    a0, a1 = lax.fori_loop(0, L, per_t, (z, z))

    c0acc = jnp.zeros((NLANE,), jnp.float32)
    c1acc = jnp.zeros((NLANE,), jnp.float32)
    for k in range(TCH):
      off = T_OFFS[k]
      iv = idx_v[pl.ds(off, NLANE)]
      m = iv == 0
      c0acc = c0acc + jnp.where(m, c0_v[pl.ds(k * NLANE, NLANE)], 0.0)
      c1acc = c1acc + jnp.where(m, c0_v[pl.ds((TCH + k) * NLANE, NLANE)], 0.0)

    log_v[b_local, 0, :] = a0 - c0acc
    log_v[b_local, 1, :] = a1 - c1acc
    return carry

  lax.fori_loop(0, BPW, per_elem, 0)
  pltpu.sync_copy(log_v, out_hbm.at[pl.ds(base, BPW)])


def _make_sc_logits():
  mesh = plsc.VectorSubcoreMesh(core_axis_name="c", subcore_axis_name="s")
  return functools.partial(
      pl.kernel,
      mesh=mesh,
      compiler_params=pltpu.CompilerParams(use_tc_tiling_on_sc=False),
      out_type=jax.ShapeDtypeStruct((B, 2, NLANE), jnp.float32),
      scratch_types=[
          pltpu.VMEM((L,), jnp.int32),
          pltpu.VMEM((L, DP), jnp.float32),
          pltpu.VMEM((2 * L * WSTRIDE,), jnp.float32),
          pltpu.VMEM((2 * TCH * NLANE,), jnp.float32),
          pltpu.VMEM((BPW, 2, NLANE), jnp.float32),
          pltpu.SemaphoreType.DMA,
          pltpu.SemaphoreType.DMA,
      ],
  )(_sc_logits_kernel)


_sc_logits = _make_sc_logits()


def _softmax_body(p_ref, b_ref, o_ref):
  x = jnp.sum(p_ref[...], axis=-1) + b_ref[...]
  m = jnp.max(x, axis=-1, keepdims=True)
  e = jnp.exp(x - m)
  o_ref[...] = (x - m) - jnp.log(jnp.sum(e, axis=-1, keepdims=True))


def _log_softmax(partials, b):
  return pl.pallas_call(
      _softmax_body,
      out_shape=jax.ShapeDtypeStruct((B, 2), jnp.float32),
  )(partials, b.reshape(1, 2))


def kernel(input, embedding, W, b):
  idx = input.astype(jnp.int32)
  table_p = jnp.pad(embedding, ((0, 0), (0, DP - D)))

  Wr = W.reshape(2, L, D)
  W4 = jnp.pad(Wr, ((0, 0), (0, 0), (0, WSTRIDE - D)))
  Wi = jnp.transpose(W4, (1, 0, 2)).reshape(-1)

  cvec = jnp.einsum("d,ctd->ct", embedding[0], Wr)
  cmain = cvec[:, :192].reshape(2, 12, NLANE)
  ctail = jnp.concatenate(
      [jnp.zeros((2, 8), cvec.dtype), cvec[:, 192:]], axis=-1
  ).reshape(2, 1, NLANE)
  c0sc = jnp.concatenate([cmain, ctail], axis=1)

  partials = _sc_logits(idx, table_p, Wi, c0sc.reshape(-1))
  return _log_softmax(partials, b)


# speedup vs baseline: 1.0295x; 1.0295x over previous
import functools

import jax
import jax.numpy as jnp
from jax import lax
from jax.experimental import pallas as pl
from jax.experimental.pallas import tpu as pltpu
from jax.experimental.pallas import tpu_sc as plsc

VOCAB = 1000000
D = 200
DW = 128
NJ = 7
L = 200
TL = 104
B = 4096
NLANE = 16
G = 4
NPAIR = 16
BPP = B // NPAIR
WPT = 2 * NJ * 2 * NLANE


def _sc_logits_kernel(idx_hbm, table_hbm, w_hbm, c0_hbm, out_hbm,
                      idx_v, rows_v, w_v, c0_v, log_v, gi_v, sem):
  wid = lax.axis_index("s") * 2 + lax.axis_index("c")
  h = wid % 2
  pair = wid // 2
  base_b = pair * BPP
  iota16 = lax.iota(jnp.int32, NLANE)

  wrow0 = h * (4 * TL)
  for cchunk in range(26):
    gi_v[pl.ds(cchunk * NLANE, NLANE)] = wrow0 + cchunk * NLANE + iota16

  wcps = []
  for cchunk in range(26):
    wcps.append(pltpu.async_copy(
        w_hbm.at[gi_v.at[pl.ds(cchunk * NLANE, NLANE)]],
        w_v.at[pl.ds(cchunk * NLANE, NLANE)], sem))
  pltpu.sync_copy(c0_hbm, c0_v)
  for cp in wcps:
    cp.wait()

  def per_sgroup(sg, carry):
    gi_v[pl.ds(416, NLANE)] = h * B + base_b + sg * 8 + iota16
    pltpu.async_copy(
        idx_hbm.at[gi_v.at[pl.ds(416, 8)]], idx_v, sem).wait()
    c = carry
    for sub in range(2):
      c = per_half_group(sg, sub, c)
    pltpu.sync_copy(log_v, out_hbm.at[h, pl.ds(base_b + sg * 8, 8)])
    return c

  def per_half_group(sg, sub, carry):
    eb = sub * G
    copies = []
    for g in range(G):
      copies.append(
          pltpu.async_copy(
              table_hbm.at[idx_v.at[sub * G + g, pl.ds(0, TL)]],
              rows_v.at[g], sem))
    for cp in copies:
      cp.wait()

    def per_t(t, accs):
      accs = list(accs)
      wr = t * 4
      for j in range(NJ):
        rj, cj = divmod(j * 32, 128)
        w0e = w_v[wr + rj, pl.ds(cj, NLANE)]
        w0o = w_v[wr + rj, pl.ds(cj + NLANE, NLANE)]
        w1e = w_v[wr + 2 + rj, pl.ds(cj, NLANE)]
        w1o = w_v[wr + 2 + rj, pl.ds(cj + NLANE, NLANE)]
        for g in range(G):
          vi = rows_v[g, t, pl.ds(j * NLANE, NLANE)]
          re = lax.bitcast_convert_type(lax.shift_left(vi, 16), jnp.float32)
          ro = lax.bitcast_convert_type(
              jnp.bitwise_and(vi, jnp.int32(-65536)), jnp.float32)
          accs[2 * g] = accs[2 * g] + re * w0e + ro * w0o
          accs[2 * g + 1] = accs[2 * g + 1] + re * w1e + ro * w1o
      return tuple(accs)

    z = jnp.zeros((NLANE,), jnp.float32)
    accs = lax.fori_loop(0, TL, per_t, (z,) * (2 * G))

    for g in range(G):
      c0acc = jnp.zeros((NLANE,), jnp.float32)
      c1acc = jnp.zeros((NLANE,), jnp.float32)
      for k in range(7):
        iv = idx_v[sub * G + g, pl.ds(k * NLANE, NLANE)]
        m = iv == 0
        hb = h * 224
        c0acc = c0acc + jnp.where(m, c0_v[pl.ds(hb + k * NLANE, NLANE)], 0.0)
        c1acc = c1acc + jnp.where(
            m, c0_v[pl.ds(hb + 112 + k * NLANE, NLANE)], 0.0)
      log_v[eb + g, 0, :] = accs[2 * g] - c0acc
      log_v[eb + g, 1, :] = accs[2 * g + 1] - c1acc
    return carry

  lax.fori_loop(0, BPP // 8, per_sgroup, 0)


def _make_sc_logits():
  mesh = plsc.VectorSubcoreMesh(core_axis_name="c", subcore_axis_name="s")
  return functools.partial(
      pl.kernel,
      mesh=mesh,
      out_type=jax.ShapeDtypeStruct((2, B, 2, NLANE), jnp.float32),
      scratch_types=[
          pltpu.VMEM((8, 1024), jnp.int32),
          pltpu.VMEM((G, TL, DW), jnp.int32),
          pltpu.VMEM((4 * TL, 128), jnp.float32),
          pltpu.VMEM((2 * 224,), jnp.float32),
          pltpu.VMEM((8, 2, NLANE), jnp.float32),
          pltpu.VMEM((432,), jnp.int32),
          pltpu.SemaphoreType.DMA,

      ],
  )(_sc_logits_kernel)


_sc_logits = _make_sc_logits()


def _softmax_body(p_ref, b_ref, o_ref):
  x = jnp.sum(p_ref[...], axis=(0, 3)) + b_ref[...]
  m = jnp.max(x, axis=-1, keepdims=True)
  e = jnp.exp(x - m)
  o_ref[...] = (x - m) - jnp.log(jnp.sum(e, axis=-1, keepdims=True))


def _log_softmax(partials, b):
  return pl.pallas_call(
      _softmax_body,
      out_shape=jax.ShapeDtypeStruct((B, 2), jnp.float32),
  )(partials, b.reshape(1, 2))


def kernel(input, embedding, W, b):
  idx = input.astype(jnp.int32)

  tb = jnp.pad(embedding, ((0, 0), (0, 256 - D))).astype(jnp.bfloat16)
  ti = jax.lax.bitcast_convert_type(tb.reshape(VOCAB, DW, 2), jnp.int32)

  Wr = W.reshape(2, L, D)
  W4 = jnp.pad(Wr, ((0, 0), (0, 0), (0, 224 - D)))
  Wd = W4.reshape(2, L, NJ, NLANE, 2)
  Wt = jnp.transpose(Wd, (1, 0, 2, 4, 3))
  Wf = jnp.pad(Wt.reshape(L, 2, 224), ((0, 0), (0, 0), (0, 32)))
  Wrow = Wf.reshape(L, 4, 128)
  half0 = Wrow[:TL]
  half1 = Wrow[96:].at[:8].set(0.0)
  Wsc = jnp.concatenate([half0, half1]).reshape(2 * TL * 4, 128)

  row0 = tb[0, :D].astype(jnp.float32)
  cvec = jnp.einsum("d,ctd->ct", row0, Wr)
  cpad = jnp.pad(cvec, ((0, 0), (0, 24)))
  ch0 = cpad[:, :112].at[:, TL:].set(0.0)
  ch1 = cpad[:, 96:208].at[:, :8].set(0.0).at[:, TL:].set(0.0)
  c0sc = jnp.stack([ch0, ch1]).reshape(2, 2 * 112)

  idx_p = jnp.pad(idx, ((0, 0), (0, 224 - L)))
  idx_sh = jnp.stack([idx_p[:, :128], idx_p[:, 96:224]]).reshape(2 * B, 128)
  idx_sh = jnp.pad(idx_sh, ((0, 0), (0, 896)))
  partials = _sc_logits(idx_sh, ti, Wsc, c0sc.reshape(-1))
  return _log_softmax(partials, b)
